# baseline (speedup 1.0000x reference)
import numpy as np
import jax
import jax.numpy as jnp
from jax.experimental import pallas as pl
from jax.experimental.pallas import tpu as pltpu

NUM_STEPS = 1000

_beta = np.linspace(0.0001, 0.02, NUM_STEPS, dtype=np.float32)
_ac = np.cumprod((1.0 - _beta).astype(np.float64)).astype(np.float32)
_SQRT_AC = np.sqrt(_ac).astype(np.float32)
_SQRT_1MAC = np.sqrt(1.0 - _ac).astype(np.float32)

_ROW = 3 * 256 * 256
_LANES = 128
_SUB = _ROW // _LANES
_CHUNKS = 4
_CSUB = _SUB // _CHUNKS


def _fma_kernel(t_ref, a_tab_ref, b_tab_ref, x_ref, n_ref, o_ref):
    bidx = pl.program_id(0)
    tv = t_ref[bidx]
    a = a_tab_ref[tv]
    b = b_tab_ref[tv]
    o_ref[...] = a * x_ref[...] + b * n_ref[...]


@jax.jit
def kernel(x_start, t, noise):
    batch = x_start.shape[0]
    x3 = x_start.reshape(batch, _SUB, _LANES)
    n3 = noise.reshape(batch, _SUB, _LANES)
    t32 = t.astype(jnp.int32)

    grid_spec = pltpu.PrefetchScalarGridSpec(
        num_scalar_prefetch=3,
        grid=(batch, _CHUNKS),
        in_specs=[
            pl.BlockSpec((1, _CSUB, _LANES), lambda i, j, *_: (i, j, 0)),
            pl.BlockSpec((1, _CSUB, _LANES), lambda i, j, *_: (i, j, 0)),
        ],
        out_specs=pl.BlockSpec((1, _CSUB, _LANES), lambda i, j, *_: (i, j, 0)),
    )

    out = pl.pallas_call(
        _fma_kernel,
        grid_spec=grid_spec,
        out_shape=jax.ShapeDtypeStruct((batch, _SUB, _LANES), jnp.float32),
    )(t32, jnp.asarray(_SQRT_AC), jnp.asarray(_SQRT_1MAC), x3, n3)
    return out.reshape(x_start.shape)


# speedup vs baseline: 6.6128x; 6.6128x over previous
import numpy as np
import jax
import jax.numpy as jnp
from jax.experimental import pallas as pl
from jax.experimental.pallas import tpu as pltpu

NUM_STEPS = 1000

_beta = np.linspace(0.0001, 0.02, NUM_STEPS, dtype=np.float32)
_ac = np.cumprod((1.0 - _beta).astype(np.float64)).astype(np.float32)
_SQRT_AC = np.sqrt(_ac).astype(np.float32)
_SQRT_1MAC = np.sqrt(1.0 - _ac).astype(np.float32)

_BBLK = 8


def _fma_kernel(t_ref, a_tab_ref, b_tab_ref, x_ref, n_ref, o_ref):
    g = pl.program_id(0)
    for k in range(_BBLK):
        tv = t_ref[g * _BBLK + k]
        a = a_tab_ref[tv]
        b = b_tab_ref[tv]
        o_ref[k] = a * x_ref[k] + b * n_ref[k]


@jax.jit
def kernel(x_start, t, noise):
    batch = x_start.shape[0]
    t32 = t.astype(jnp.int32)

    grid_spec = pltpu.PrefetchScalarGridSpec(
        num_scalar_prefetch=3,
        grid=(batch // _BBLK,),
        in_specs=[
            pl.BlockSpec((_BBLK, 3, 256, 256), lambda i, *_: (i, 0, 0, 0)),
            pl.BlockSpec((_BBLK, 3, 256, 256), lambda i, *_: (i, 0, 0, 0)),
        ],
        out_specs=pl.BlockSpec((_BBLK, 3, 256, 256), lambda i, *_: (i, 0, 0, 0)),
    )

    return pl.pallas_call(
        _fma_kernel,
        grid_spec=grid_spec,
        out_shape=jax.ShapeDtypeStruct(x_start.shape, jnp.float32),
        compiler_params=pltpu.CompilerParams(
            dimension_semantics=("arbitrary",),
            vmem_limit_bytes=100 * 1024 * 1024,
        ),
    )(t32, jnp.asarray(_SQRT_AC), jnp.asarray(_SQRT_1MAC), x_start, noise)
